# baseline (speedup 1.0000x reference)
import functools

import jax
import jax.numpy as jnp
from jax import lax
from jax.experimental import pallas as pl
from jax.experimental.pallas import tpu as pltpu
from jax.experimental.pallas import tpu_sc as plsc

N_NODES = 10000
DEG = 32
EMBED = 128
NW = 32
SLICE = 128
NBUF = 6

NCHUNK = 1
CH = N_NODES // NCHUNK
E_PER_W = CH * DEG // NW
E_FULL = E_PER_W // SLICE
E_TAIL = E_PER_W - E_FULL * SLICE
U_PER_W = -(-CH // NW // SLICE + 0) * SLICE if CH // NW else SLICE
U_PER_W = ((CH + NW - 1) // NW + SLICE - 1) // SLICE * SLICE
U_SLICES = U_PER_W // SLICE
U_ROWS = NW * U_PER_W
E_SLICES = E_FULL + (1 if E_TAIL else 0)
IDX_SLICES = E_SLICES + U_SLICES
RING_ITERS = E_FULL // NBUF

NB = 200


def _gather_body(table, idx, out_e, out_u, idx_v, *rest):
    bufs = rest[:NBUF]
    gs = rest[NBUF:2 * NBUF]
    ss = rest[2 * NBUF:3 * NBUF]
    wid = lax.axis_index("s") * 2 + lax.axis_index("c")
    e_base = wid * E_PER_W
    u_base = wid * U_PER_W

    pltpu.sync_copy(idx.at[wid], idx_v)

    def g_start(s, b):
        pltpu.async_copy(table.at[idx_v.at[s]], bufs[b], gs[b])

    def g_wait(b):
        pltpu.make_async_copy(table.at[idx_v.at[0]], bufs[b], gs[b]).wait()

    def s_start(s, b):
        pltpu.async_copy(bufs[b], out_e.at[pl.ds(e_base + s * SLICE, SLICE)],
                         ss[b])

    def s_wait(b):
        pltpu.make_async_copy(bufs[b],
                              out_e.at[pl.ds(e_base, SLICE)], ss[b]).wait()

    for b in range(NBUF):
        g_start(b, b)

    def body(g, carry):
        for b in range(NBUF):
            g_wait(b)
            s_start(NBUF * g + b, b)
        for b in range(NBUF):
            s_wait(b)
            g_start(NBUF * (g + 1) + b, b)
        return carry

    if RING_ITERS > 1:
        lax.fori_loop(0, RING_ITERS - 1, body, 0)

    gl = RING_ITERS - 1
    for b in range(NBUF):
        g_wait(b)
        s_start(NBUF * gl + b, b)

    extra = list(range(NBUF * RING_ITERS, IDX_SLICES))
    for j, s in enumerate(extra):
        b = j % NBUF
        s_wait(b)
        g_start(s, b)
    for b in range(len(extra), NBUF):
        s_wait(b)

    for j, s in enumerate(extra):
        b = j % NBUF
        g_wait(b)
        if s < E_FULL:
            pltpu.sync_copy(bufs[b], out_e.at[pl.ds(e_base + s * SLICE, SLICE)])
        elif s < E_SLICES:
            pltpu.sync_copy(bufs[b].at[pl.ds(0, E_TAIL)],
                            out_e.at[pl.ds(e_base + s * SLICE, E_TAIL)])
        else:
            k = s - E_SLICES
            pltpu.sync_copy(bufs[b], out_u.at[pl.ds(u_base + k * SLICE, SLICE)])


def _sc_gather(table, idx):
    mesh = plsc.VectorSubcoreMesh(core_axis_name="c", subcore_axis_name="s")
    fn = functools.partial(
        pl.kernel,
        mesh=mesh,
        out_type=[
            jax.ShapeDtypeStruct((CH * DEG, EMBED), jnp.float32),
            jax.ShapeDtypeStruct((U_ROWS, EMBED), jnp.float32),
        ],
        scratch_types=(
            [pltpu.VMEM((IDX_SLICES, SLICE), jnp.int32)]
            + [pltpu.VMEM((SLICE, EMBED), jnp.float32) for _ in range(NBUF)]
            + [pltpu.SemaphoreType.DMA for _ in range(2 * NBUF)]
        ),
    )(_gather_body)
    return fn(table, idx)


def _mlp_body(e_ref, u_ref, w1a_ref, w1b_ref, b1_ref, w2_ref, b2_ref,
              w3_ref, o_ref):
    e2 = e_ref[...]
    u = u_ref[...]
    bsum = jnp.dot(u, w1b_ref[...], preferred_element_type=jnp.float32)
    bsum = bsum + b1_ref[...]
    bex = jnp.broadcast_to(bsum[:, None, :], (NB, DEG, EMBED))
    bex = bex.reshape(NB * DEG, EMBED)
    h1 = jnp.dot(e2, w1a_ref[...], preferred_element_type=jnp.float32) + bex
    h1 = jnp.maximum(h1, 0.0)
    h2 = jnp.dot(h1, w2_ref[...], preferred_element_type=jnp.float32)
    h2 = jnp.maximum(h2 + b2_ref[...], 0.0)
    s = jnp.dot(h2, w3_ref[...], preferred_element_type=jnp.float32)
    s3 = s.reshape(NB, DEG, 1)
    m = jnp.max(s3, axis=1, keepdims=True)
    p = jnp.exp(s3 - m)
    att = p / jnp.sum(p, axis=1, keepdims=True)
    e3 = e2.reshape(NB, DEG, EMBED)
    o_ref[...] = jnp.sum(e3 * att, axis=1)


def _tc_mlp(eg, ug, W1a, W1b, b1, W2, b2, W3):
    grid = (CH // NB,)
    return pl.pallas_call(
        _mlp_body,
        grid=grid,
        in_specs=[
            pl.BlockSpec((NB * DEG, EMBED), lambda i: (i, 0)),
            pl.BlockSpec((NB, EMBED), lambda i: (i, 0)),
            pl.BlockSpec((EMBED, EMBED), lambda i: (0, 0)),
            pl.BlockSpec((EMBED, EMBED), lambda i: (0, 0)),
            pl.BlockSpec((1, EMBED), lambda i: (0, 0)),
            pl.BlockSpec((EMBED, EMBED), lambda i: (0, 0)),
            pl.BlockSpec((1, EMBED), lambda i: (0, 0)),
            pl.BlockSpec((EMBED, 1), lambda i: (0, 0)),
        ],
        out_specs=pl.BlockSpec((NB, EMBED), lambda i: (i, 0)),
        out_shape=jax.ShapeDtypeStruct((CH, EMBED), jnp.float32),
    )(eg, ug, W1a, W1b, b1, W2, b2, W3)


def kernel(nodes, to_neighs, u2e_weight, W1, b1, W2, b2, W3, b3):
    W1a = W1[:EMBED, :]
    W1b = W1[EMBED:, :]
    b1r = b1.reshape(1, EMBED)
    b2r = b2.reshape(1, EMBED)

    outs = []
    for c in range(NCHUNK):
        nodes_c = lax.dynamic_slice_in_dim(nodes, c * CH, CH)
        tn_c = lax.dynamic_slice_in_dim(to_neighs, c * CH, CH)
        e_idx = tn_c.reshape(NW, E_PER_W)
        if E_TAIL:
            e_idx = jnp.pad(e_idx, ((0, 0), (0, SLICE - E_TAIL)))
        u_idx = jnp.pad(nodes_c, (0, U_ROWS - CH)).reshape(NW, U_PER_W)
        idx = jnp.concatenate([e_idx, u_idx], axis=1).reshape(
            NW, IDX_SLICES, SLICE)
        idx = jnp.zeros_like(idx)

        eg, ug = _sc_gather(u2e_weight, idx)
        outs.append(_tc_mlp(eg, ug, W1a, W1b, b1r, W2, b2r, W3))

    del b3
    return jnp.concatenate(outs, axis=0)


# speedup vs baseline: 26.0188x; 26.0188x over previous
import functools

import jax
import jax.numpy as jnp
from jax import lax
from jax.experimental import pallas as pl
from jax.experimental.pallas import tpu as pltpu
from jax.experimental.pallas import tpu_sc as plsc

N_NODES = 10000
DEG = 32
EMBED = 128
NW = 32
SLICE = 128
NBUF = 6

NCHUNK = 1
CH = N_NODES // NCHUNK
E_PER_W = CH * DEG // NW
E_FULL = E_PER_W // SLICE
E_TAIL = E_PER_W - E_FULL * SLICE
U_PER_W = -(-CH // NW // SLICE + 0) * SLICE if CH // NW else SLICE
U_PER_W = ((CH + NW - 1) // NW + SLICE - 1) // SLICE * SLICE
U_SLICES = U_PER_W // SLICE
U_ROWS = NW * U_PER_W
E_SLICES = E_FULL + (1 if E_TAIL else 0)
IDX_SLICES = E_SLICES + U_SLICES
RING_ITERS = E_FULL // NBUF

NB = 400


def _gather_body(table, idx, out_e, out_u, idx_v, *rest):
    bufs = rest[:NBUF]
    gs = rest[NBUF:2 * NBUF]
    ss = rest[2 * NBUF:3 * NBUF]
    wid = lax.axis_index("s") * 2 + lax.axis_index("c")
    e_base = wid * E_PER_W
    u_base = wid * U_PER_W

    pltpu.sync_copy(idx.at[wid], idx_v)

    def g_start(s, b):
        pltpu.async_copy(table.at[idx_v.at[s]], bufs[b], gs[b])

    def g_wait(b):
        pltpu.make_async_copy(table.at[idx_v.at[0]], bufs[b], gs[b]).wait()

    def s_start(s, b):
        pltpu.async_copy(bufs[b], out_e.at[pl.ds(e_base + s * SLICE, SLICE)],
                         ss[b])

    def s_wait(b):
        pltpu.make_async_copy(bufs[b],
                              out_e.at[pl.ds(e_base, SLICE)], ss[b]).wait()

    for b in range(NBUF):
        g_start(b, b)

    def body(g, carry):
        for b in range(NBUF):
            g_wait(b)
            s_start(NBUF * g + b, b)
        for b in range(NBUF):
            s_wait(b)
            g_start(NBUF * (g + 1) + b, b)
        return carry

    if RING_ITERS > 1:
        lax.fori_loop(0, RING_ITERS - 1, body, 0)

    gl = RING_ITERS - 1
    for b in range(NBUF):
        g_wait(b)
        s_start(NBUF * gl + b, b)

    extra = list(range(NBUF * RING_ITERS, IDX_SLICES))
    for j, s in enumerate(extra):
        b = j % NBUF
        s_wait(b)
        g_start(s, b)
    for b in range(len(extra), NBUF):
        s_wait(b)

    for j, s in enumerate(extra):
        b = j % NBUF
        g_wait(b)
        if s < E_FULL:
            pltpu.sync_copy(bufs[b], out_e.at[pl.ds(e_base + s * SLICE, SLICE)])
        elif s < E_SLICES:
            pltpu.sync_copy(bufs[b].at[pl.ds(0, E_TAIL)],
                            out_e.at[pl.ds(e_base + s * SLICE, E_TAIL)])
        else:
            k = s - E_SLICES
            pltpu.sync_copy(bufs[b], out_u.at[pl.ds(u_base + k * SLICE, SLICE)])


def _sc_gather(table, idx):
    mesh = plsc.VectorSubcoreMesh(core_axis_name="c", subcore_axis_name="s")
    fn = functools.partial(
        pl.kernel,
        mesh=mesh,
        out_type=[
            jax.ShapeDtypeStruct((CH * DEG, EMBED), jnp.float32),
            jax.ShapeDtypeStruct((U_ROWS, EMBED), jnp.float32),
        ],
        scratch_types=(
            [pltpu.VMEM((IDX_SLICES, SLICE), jnp.int32)]
            + [pltpu.VMEM((SLICE, EMBED), jnp.float32) for _ in range(NBUF)]
            + [pltpu.SemaphoreType.DMA for _ in range(2 * NBUF)]
        ),
    )(_gather_body)
    return fn(table, idx)


def _mlp_body(e_ref, u_ref, w1a_ref, w1b_ref, b1_ref, w2_ref, b2_ref,
              w3_ref, o_ref):
    e2 = e_ref[...]
    u = u_ref[...]
    bsum = jnp.dot(u, w1b_ref[...], preferred_element_type=jnp.float32)
    bsum = bsum + b1_ref[...]
    bex = jnp.broadcast_to(bsum[:, None, :], (NB, DEG, EMBED))
    bex = bex.reshape(NB * DEG, EMBED)
    h1 = jnp.dot(e2, w1a_ref[...], preferred_element_type=jnp.float32) + bex
    h1 = jnp.maximum(h1, 0.0)
    h2 = jnp.dot(h1, w2_ref[...], preferred_element_type=jnp.float32)
    h2 = jnp.maximum(h2 + b2_ref[...], 0.0)
    s = jnp.dot(h2, w3_ref[...], preferred_element_type=jnp.float32)
    s3 = s.reshape(NB, DEG, 1)
    m = jnp.max(s3, axis=1, keepdims=True)
    p = jnp.exp(s3 - m)
    att = p / jnp.sum(p, axis=1, keepdims=True)
    e3 = e2.reshape(NB, DEG, EMBED)
    o_ref[...] = jnp.sum(e3 * att, axis=1)


def _tc_mlp(eg, ug, W1a, W1b, b1, W2, b2, W3):
    grid = (CH // NB,)
    return pl.pallas_call(
        _mlp_body,
        grid=grid,
        in_specs=[
            pl.BlockSpec((NB * DEG, EMBED), lambda i: (i, 0)),
            pl.BlockSpec((NB, EMBED), lambda i: (i, 0)),
            pl.BlockSpec((EMBED, EMBED), lambda i: (0, 0)),
            pl.BlockSpec((EMBED, EMBED), lambda i: (0, 0)),
            pl.BlockSpec((1, EMBED), lambda i: (0, 0)),
            pl.BlockSpec((EMBED, EMBED), lambda i: (0, 0)),
            pl.BlockSpec((1, EMBED), lambda i: (0, 0)),
            pl.BlockSpec((EMBED, 1), lambda i: (0, 0)),
        ],
        out_specs=pl.BlockSpec((NB, EMBED), lambda i: (i, 0)),
        out_shape=jax.ShapeDtypeStruct((CH, EMBED), jnp.float32),
    )(eg, ug, W1a, W1b, b1, W2, b2, W3)


def kernel(nodes, to_neighs, u2e_weight, W1, b1, W2, b2, W3, b3):
    W1a = W1[:EMBED, :]
    W1b = W1[EMBED:, :]
    b1r = b1.reshape(1, EMBED)
    b2r = b2.reshape(1, EMBED)

    outs = []
    for c in range(NCHUNK):
        nodes_c = lax.dynamic_slice_in_dim(nodes, c * CH, CH)
        tn_c = lax.dynamic_slice_in_dim(to_neighs, c * CH, CH)
        e_idx = tn_c.reshape(NW, E_PER_W)
        if E_TAIL:
            e_idx = jnp.pad(e_idx, ((0, 0), (0, SLICE - E_TAIL)))
        u_idx = jnp.pad(nodes_c, (0, U_ROWS - CH)).reshape(NW, U_PER_W)
        idx = jnp.concatenate([e_idx, u_idx], axis=1).reshape(
            NW, IDX_SLICES, SLICE)

        eg, ug = _sc_gather(u2e_weight, idx)
        outs.append(_tc_mlp(eg, ug, W1a, W1b, b1r, W2, b2r, W3))

    del b3
    return jnp.concatenate(outs, axis=0)
